# baseline (speedup 1.0000x reference)
import functools
import math

import jax
import jax.numpy as jnp
from jax import lax
from jax.experimental import pallas as pl
from jax.experimental.pallas import tpu as pltpu
from jax.experimental.pallas import tpu_sc as plsc

B, H, N, D = 4, 8, 1024, 256
NHID = 256
BH = B * H
K = (N * N) // int(math.sqrt(N))

_SQRT_HALF = 0.7071067811865476

_NC, _NS = 2, 16
_CROWS = 16
_NCHUNK = N // _CROWS
_NBINS = 32768


def _select_from_top(hist, kleft, base=0, nbins=_NBINS):
    iota = lax.iota(jnp.int32, 16)
    big = jnp.float32(3e9)

    def cond(st):
        g, acc, q, cnt, found = st
        return jnp.logical_and(jnp.logical_not(found), g >= 0)

    def body(st):
        g, acc, q, cnt, found = st
        vec = hist[pl.ds(base + g * 16, 16)]
        rv = lax.rev(vec, (0,))
        cs = plsc.cumsum(rv)
        tot = acc + cs
        mask = tot >= kleft
        nset = jnp.sum(jnp.where(mask, 1, 0))
        has = nset > 0
        lstar = 16 - nset
        csl = jnp.min(jnp.where(mask, cs, big))
        rvl = jnp.sum(jnp.where(iota == lstar, rv, 0.0))
        q_new = g * 16 + 15 - lstar
        cnt_new = acc + csl - rvl
        acc_new = acc + jnp.sum(vec)
        return (
            g - 1,
            jnp.where(has, acc, acc_new),
            jnp.where(has, q_new, q),
            jnp.where(has, cnt_new, cnt),
            has,
        )

    st = (
        jnp.int32(nbins // 16 - 1),
        jnp.float32(0.0),
        jnp.int32(0),
        jnp.float32(0.0),
        jnp.bool_(False),
    )
    g, acc, q, cnt, found = lax.while_loop(cond, body, st)
    return q, cnt


def _zero_hist(hist):
    zf = jnp.zeros((16,), jnp.float32)

    @plsc.parallel_loop(0, _NBINS, step=16, unroll=8)
    def _(i):
        hist[pl.ds(i, 16)] = zf


def _stream_pass(adj_hbm, wid, bufs, sems, hist, binfn):
    nbuf = len(bufs)
    onesf = jnp.ones((16,), jnp.float32)
    for b in range(nbuf):
        pltpu.async_copy(
            adj_hbm.at[wid, pl.ds(b * _CROWS, _CROWS)], bufs[b], sems[b]
        )

    def outer(g, carry):
        for bsl in range(nbuf):
            c = g * nbuf + bsl
            buf, sem = bufs[bsl], sems[bsl]
            pltpu.make_async_copy(
                adj_hbm.at[wid, pl.ds(c * _CROWS, _CROWS)], buf, sem
            ).wait()

            @plsc.parallel_loop(0, _CROWS * N, step=32, unroll=8)
            def _(i):
                row = lax.shift_right_logical(i, 10)
                col = jnp.bitwise_and(i, N - 1)
                for h in range(2):
                    v = buf[row, pl.ds(col + 16 * h, 16)]
                    pat = lax.bitcast_convert_type(v, jnp.int32)
                    b2, m = binfn(pat)
                    plsc.addupdate_scatter(hist, [b2], onesf, mask=m)

            @pl.when(g < _NCHUNK // nbuf - 1)
            def _():
                pltpu.async_copy(
                    adj_hbm.at[wid, pl.ds((c + nbuf) * _CROWS, _CROWS)],
                    buf, sem,
                )
        return carry

    lax.fori_loop(0, _NCHUNK // len(bufs), outer, 0)


def _thresh_body(adj_hbm, thr_hbm, buf0, buf1, buf2, buf3, hist, tbuf,
                 sem0, sem1, sem2, sem3):
    wid = lax.axis_index("s") * _NC + lax.axis_index("c")
    bufs, sems = (buf0, buf1, buf2, buf3), (sem0, sem1, sem2, sem3)

    _zero_hist(hist)
    _stream_pass(
        adj_hbm, wid, bufs, sems, hist,
        lambda pat: (lax.shift_right_logical(pat, 15), None),
    )
    q1, cnt1 = _select_from_top(hist, jnp.float32(K))

    _zero_hist(hist)
    _stream_pass(
        adj_hbm, wid, bufs, sems, hist,
        lambda pat: (
            jnp.bitwise_and(pat, 0x7FFF),
            lax.shift_right_logical(pat, 15) == q1,
        ),
    )
    q2, _ = _select_from_top(hist, jnp.float32(K) - cnt1)

    t_pat = jnp.left_shift(q1, 15) | q2
    tbuf[...] = jnp.zeros((16,), jnp.int32) + t_pat
    pltpu.sync_copy(tbuf, thr_hbm.at[wid])


_sc_thresholds = functools.partial(
    pl.kernel,
    mesh=plsc.VectorSubcoreMesh(core_axis_name="c", subcore_axis_name="s"),
    out_type=jax.ShapeDtypeStruct((BH, 16), jnp.int32),
    scratch_types=[
        pltpu.VMEM((_CROWS, N), jnp.float32),
        pltpu.VMEM((_CROWS, N), jnp.float32),
        pltpu.VMEM((_CROWS, N), jnp.float32),
        pltpu.VMEM((_CROWS, N), jnp.float32),
        pltpu.VMEM((_NBINS,), jnp.float32),
        pltpu.VMEM((16,), jnp.int32),
        pltpu.SemaphoreType.DMA,
        pltpu.SemaphoreType.DMA,
        pltpu.SemaphoreType.DMA,
        pltpu.SemaphoreType.DMA,
    ],
    compiler_params=pltpu.CompilerParams(needs_layout_passes=False),
)(_thresh_body)


def _gin_kernel(thr_ref, adj_ref, x_ref, w_ref, b_ref, eps_ref, out_ref, adjn_ref):
    i = pl.program_id(0)
    a = adj_ref[0]
    thresh = thr_ref[i, 0]

    m = jnp.where(a >= thresh, a, 0.0)
    mt = m.T
    sym = (m + mt) * 0.5
    deg = jnp.sum(sym, axis=1)
    pos = deg > 0.0
    dinv = jnp.where(pos, lax.rsqrt(jnp.where(pos, deg, 1.0)), 0.0)
    normed = sym * dinv[:, None] * dinv[None, :]
    adjn_ref[0] = normed

    x = x_ref[0]
    agg = lax.dot(normed, x)
    eps = eps_ref[0, 0]
    h = (1.0 + eps) * x + agg
    z = lax.dot(h, w_ref[...])
    z = z + b_ref[0]
    out_ref[0] = 0.5 * z * (1.0 + lax.erf(z * _SQRT_HALF))


def kernel(x, adj, W, b, eps):
    xf = x.reshape(BH, N, D)
    adjf = adj.reshape(BH, N, N)
    eps2d = jnp.reshape(eps, (1, 1)).astype(jnp.float32)
    b2d = jnp.reshape(b, (1, NHID))

    thr_pat = _sc_thresholds(adjf)
    thr = lax.bitcast_convert_type(thr_pat, jnp.float32)

    out, adjn = pl.pallas_call(
        _gin_kernel,
        grid=(BH,),
        in_specs=[
            pl.BlockSpec(memory_space=pltpu.SMEM),
            pl.BlockSpec((1, N, N), lambda i: (i, 0, 0)),
            pl.BlockSpec((1, N, D), lambda i: (i, 0, 0)),
            pl.BlockSpec((D, NHID), lambda i: (0, 0)),
            pl.BlockSpec((1, NHID), lambda i: (0, 0)),
            pl.BlockSpec(memory_space=pltpu.SMEM),
        ],
        out_specs=[
            pl.BlockSpec((1, N, NHID), lambda i: (i, 0, 0)),
            pl.BlockSpec((1, N, N), lambda i: (i, 0, 0)),
        ],
        out_shape=[
            jax.ShapeDtypeStruct((BH, N, NHID), jnp.float32),
            jax.ShapeDtypeStruct((BH, N, N), jnp.float32),
        ],
    )(thr, adjf, xf, W, b2d, eps2d)

    return out.reshape(B, H, N, NHID), adjn


# speedup vs baseline: 1.0031x; 1.0031x over previous
import functools
import math

import jax
import jax.numpy as jnp
from jax import lax
from jax.experimental import pallas as pl
from jax.experimental.pallas import tpu as pltpu
from jax.experimental.pallas import tpu_sc as plsc

B, H, N, D = 4, 8, 1024, 256
NHID = 256
BH = B * H
K = (N * N) // int(math.sqrt(N))

_SQRT_HALF = 0.7071067811865476

_NC, _NS = 2, 16
_CROWS = 32
_NCHUNK = N // _CROWS
_NBINS = 32768


def _select_from_top(hist, kleft, base=0, nbins=_NBINS):
    iota = lax.iota(jnp.int32, 16)
    big = jnp.float32(3e9)

    def cond(st):
        g, acc, q, cnt, found = st
        return jnp.logical_and(jnp.logical_not(found), g >= 0)

    def body(st):
        g, acc, q, cnt, found = st
        vec = hist[pl.ds(base + g * 16, 16)]
        rv = lax.rev(vec, (0,))
        cs = plsc.cumsum(rv)
        tot = acc + cs
        mask = tot >= kleft
        nset = jnp.sum(jnp.where(mask, 1, 0))
        has = nset > 0
        lstar = 16 - nset
        csl = jnp.min(jnp.where(mask, cs, big))
        rvl = jnp.sum(jnp.where(iota == lstar, rv, 0.0))
        q_new = g * 16 + 15 - lstar
        cnt_new = acc + csl - rvl
        acc_new = acc + jnp.sum(vec)
        return (
            g - 1,
            jnp.where(has, acc, acc_new),
            jnp.where(has, q_new, q),
            jnp.where(has, cnt_new, cnt),
            has,
        )

    st = (
        jnp.int32(nbins // 16 - 1),
        jnp.float32(0.0),
        jnp.int32(0),
        jnp.float32(0.0),
        jnp.bool_(False),
    )
    g, acc, q, cnt, found = lax.while_loop(cond, body, st)
    return q, cnt


def _zero_hist(hist):
    zf = jnp.zeros((16,), jnp.float32)

    @plsc.parallel_loop(0, _NBINS, step=16, unroll=8)
    def _(i):
        hist[pl.ds(i, 16)] = zf


def _stream_pass(adj_hbm, wid, bufs, sems, hist, binfn):
    nbuf = len(bufs)
    onesf = jnp.ones((16,), jnp.float32)
    for b in range(nbuf):
        pltpu.async_copy(
            adj_hbm.at[wid, pl.ds(b * _CROWS, _CROWS)], bufs[b], sems[b]
        )

    def outer(g, carry):
        for bsl in range(nbuf):
            c = g * nbuf + bsl
            buf, sem = bufs[bsl], sems[bsl]
            pltpu.make_async_copy(
                adj_hbm.at[wid, pl.ds(c * _CROWS, _CROWS)], buf, sem
            ).wait()

            @plsc.parallel_loop(0, _CROWS * N, step=32, unroll=8)
            def _(i):
                row = lax.shift_right_logical(i, 10)
                col = jnp.bitwise_and(i, N - 1)
                for h in range(2):
                    v = buf[row, pl.ds(col + 16 * h, 16)]
                    pat = lax.bitcast_convert_type(v, jnp.int32)
                    b2, m = binfn(pat)
                    plsc.addupdate_scatter(hist, [b2], onesf, mask=m)

            @pl.when(g < _NCHUNK // nbuf - 1)
            def _():
                pltpu.async_copy(
                    adj_hbm.at[wid, pl.ds((c + nbuf) * _CROWS, _CROWS)],
                    buf, sem,
                )
        return carry

    lax.fori_loop(0, _NCHUNK // len(bufs), outer, 0)


def _thresh_body(adj_hbm, thr_hbm, buf0, buf1, hist, tbuf, sem0, sem1):
    wid = lax.axis_index("s") * _NC + lax.axis_index("c")
    bufs, sems = (buf0, buf1), (sem0, sem1)

    _zero_hist(hist)
    _stream_pass(
        adj_hbm, wid, bufs, sems, hist,
        lambda pat: (lax.shift_right_logical(pat, 15), None),
    )
    q1, cnt1 = _select_from_top(hist, jnp.float32(K))

    _zero_hist(hist)
    _stream_pass(
        adj_hbm, wid, bufs, sems, hist,
        lambda pat: (
            jnp.bitwise_and(pat, 0x7FFF),
            lax.shift_right_logical(pat, 15) == q1,
        ),
    )
    q2, _ = _select_from_top(hist, jnp.float32(K) - cnt1)

    t_pat = jnp.left_shift(q1, 15) | q2
    tbuf[...] = jnp.zeros((16,), jnp.int32) + t_pat
    pltpu.sync_copy(tbuf, thr_hbm.at[wid])


_sc_thresholds = functools.partial(
    pl.kernel,
    mesh=plsc.VectorSubcoreMesh(core_axis_name="c", subcore_axis_name="s"),
    out_type=jax.ShapeDtypeStruct((BH, 16), jnp.int32),
    scratch_types=[
        pltpu.VMEM((_CROWS, N), jnp.float32),
        pltpu.VMEM((_CROWS, N), jnp.float32),
        pltpu.VMEM((_NBINS,), jnp.float32),
        pltpu.VMEM((16,), jnp.int32),
        pltpu.SemaphoreType.DMA,
        pltpu.SemaphoreType.DMA,
    ],
    compiler_params=pltpu.CompilerParams(needs_layout_passes=False),
)(_thresh_body)


def _gin_kernel(thr_ref, adj_ref, x_ref, w_ref, b_ref, eps_ref, out_ref, adjn_ref):
    i = pl.program_id(0)
    a = adj_ref[0]
    thresh = thr_ref[i, 0]

    m = jnp.where(a >= thresh, a, 0.0)
    mt = m.T
    sym = (m + mt) * 0.5
    deg = jnp.sum(sym, axis=1)
    pos = deg > 0.0
    dinv = jnp.where(pos, lax.rsqrt(jnp.where(pos, deg, 1.0)), 0.0)
    normed = sym * dinv[:, None] * dinv[None, :]
    adjn_ref[0] = normed

    x = x_ref[0]
    agg = lax.dot(normed, x)
    eps = eps_ref[0, 0]
    h = (1.0 + eps) * x + agg
    z = lax.dot(h, w_ref[...])
    z = z + b_ref[0]
    out_ref[0] = 0.5 * z * (1.0 + lax.erf(z * _SQRT_HALF))


def kernel(x, adj, W, b, eps):
    xf = x.reshape(BH, N, D)
    adjf = adj.reshape(BH, N, N)
    eps2d = jnp.reshape(eps, (1, 1)).astype(jnp.float32)
    b2d = jnp.reshape(b, (1, NHID))

    thr_pat = _sc_thresholds(adjf)
    thr = lax.bitcast_convert_type(thr_pat, jnp.float32)

    out, adjn = pl.pallas_call(
        _gin_kernel,
        grid=(BH,),
        in_specs=[
            pl.BlockSpec(memory_space=pltpu.SMEM),
            pl.BlockSpec((1, N, N), lambda i: (i, 0, 0)),
            pl.BlockSpec((1, N, D), lambda i: (i, 0, 0)),
            pl.BlockSpec((D, NHID), lambda i: (0, 0)),
            pl.BlockSpec((1, NHID), lambda i: (0, 0)),
            pl.BlockSpec(memory_space=pltpu.SMEM),
        ],
        out_specs=[
            pl.BlockSpec((1, N, NHID), lambda i: (i, 0, 0)),
            pl.BlockSpec((1, N, N), lambda i: (i, 0, 0)),
        ],
        out_shape=[
            jax.ShapeDtypeStruct((BH, N, NHID), jnp.float32),
            jax.ShapeDtypeStruct((BH, N, N), jnp.float32),
        ],
    )(thr, adjf, xf, W, b2d, eps2d)

    return out.reshape(B, H, N, NHID), adjn
